# baseline (speedup 1.0000x reference)
import functools

import jax
import jax.numpy as jnp
from jax import lax
from jax.experimental import pallas as pl
from jax.experimental.pallas import tpu as pltpu
from jax.experimental.pallas import tpu_sc as plsc

OUT_H = 7
OUT_W = 7
SPATIAL_SCALE = 0.0625
SAMPLING_RATIO = 2

B, C, H, W = 4, 256, 64, 64
N_ROIS = 512
NWIN = 17
NWORK = 32
RPW = N_ROIS // NWORK
CCHUNKS = C // 16


def _axis_taps(coord, spacing, size):
    ii = lax.iota(jnp.int32, 16).astype(jnp.float32) + 0.5
    V = coord + ii * spacing
    valid = (V > -1.0) & (V < float(size))
    v = jnp.maximum(V, 0.0)
    l0 = v.astype(jnp.int32)
    cond = l0 >= size - 1
    lo = jnp.where(cond, size - 1, l0)
    hi = jnp.where(cond, size - 1, l0 + 1)
    vf = jnp.where(cond, float(size - 1), v)
    lw = vf - lo.astype(jnp.float32)
    wlo = jnp.where(valid, (1.0 - lw) * 0.5, 0.0)
    whi = jnp.where(valid, lw * 0.5, 0.0)
    return lo, hi, wlo, whi


def _taps(roiv, r):
    row = roiv[r]
    bi = row[0].astype(jnp.int32)
    x1 = row[1] * SPATIAL_SCALE
    y1 = row[2] * SPATIAL_SCALE
    x2 = row[3] * SPATIAL_SCALE
    y2 = row[4] * SPATIAL_SCALE
    sw = jnp.maximum(x2 - x1, 1.0) * (1.0 / (OUT_W * SAMPLING_RATIO))
    sh = jnp.maximum(y2 - y1, 1.0) * (1.0 / (OUT_H * SAMPLING_RATIO))

    lox, hix, wlox, whix = _axis_taps(x1, sw, W)
    loy, hiy, wloy, whiy = _axis_taps(y1, sh, H)

    x0 = jnp.minimum(lox[0], W - NWIN)
    y0 = jnp.minimum(loy[0], H - NWIN)
    return (bi * H + y0, x0,
            lox - x0, hix - x0, loy - y0, hiy - y0,
            wlox, whix, wloy, whiy)


def _body(feat_hbm, rois_hbm, out_hbm, roiv, tile, tmp, acc, sem, osem):
    cid = lax.axis_index("c")
    sid = lax.axis_index("s")
    wid = sid * 2 + cid
    roi0 = wid * RPW
    pltpu.sync_copy(rois_hbm.at[pl.ds(roi0, RPW)], roiv)

    def issue_window(t):
        bh0, x0 = t[0], t[1]
        pltpu.async_copy(
            feat_hbm.at[pl.ds(bh0, NWIN), pl.ds(x0, NWIN)], tile, sem)

    t0 = _taps(roiv, 0)
    issue_window(t0)

    def do_roi(r, t_cur):
        (bh0, x0, jxl, jxh, jyl, jyh, wlox, whix, wloy, whiy) = t_cur

        pltpu.make_async_copy(
            feat_hbm.at[pl.ds(0, NWIN), pl.ds(0, NWIN)], tile, sem).wait()

        xtaps = [(jxl[2 * ox], jxh[2 * ox], jxl[2 * ox + 1], jxh[2 * ox + 1],
                  wlox[2 * ox], whix[2 * ox], wlox[2 * ox + 1],
                  whix[2 * ox + 1]) for ox in range(OUT_W)]

        @plsc.parallel_loop(0, NWIN)
        def yb(y):
            for ox, (j0, j1, j2, j3, w0, w1, w2, w3) in enumerate(xtaps):
                for cc in range(CCHUNKS):
                    sl = pl.ds(cc * 16, 16)
                    tmp[y, ox, sl] = ((tile[y, j0, sl] * w0
                                       + tile[y, j1, sl] * w1)
                                      + (tile[y, j2, sl] * w2
                                         + tile[y, j3, sl] * w3))

        t_next = _taps(roiv, jnp.minimum(r + 1, RPW - 1))

        @pl.when(r + 1 < RPW)
        def _():
            issue_window(t_next)

        @pl.when(r >= 1)
        def _():
            pltpu.make_async_copy(acc, out_hbm.at[roi0], osem).wait()

        ytaps = [(jyl[2 * oy], jyh[2 * oy], jyl[2 * oy + 1], jyh[2 * oy + 1],
                  wloy[2 * oy], whiy[2 * oy], wloy[2 * oy + 1],
                  whiy[2 * oy + 1]) for oy in range(OUT_H)]

        @plsc.parallel_loop(0, OUT_W)
        def xb(ox):
            for oy, (k0, k1, k2, k3, u0, u1, u2, u3) in enumerate(ytaps):
                for cc in range(CCHUNKS):
                    sl = pl.ds(cc * 16, 16)
                    acc[oy * OUT_W + ox, sl] = ((tmp[k0, ox, sl] * u0
                                                 + tmp[k1, ox, sl] * u1)
                                                + (tmp[k2, ox, sl] * u2
                                                   + tmp[k3, ox, sl] * u3))

        pltpu.async_copy(acc, out_hbm.at[roi0 + r], osem)
        return t_next

    lax.fori_loop(0, RPW, do_roi, t0)
    pltpu.make_async_copy(acc, out_hbm.at[roi0], osem).wait()


@jax.jit
def _roialign_sc(feat2d, rois_pad):
    mesh = plsc.VectorSubcoreMesh(core_axis_name="c", subcore_axis_name="s")
    f = pl.kernel(
        _body,
        mesh=mesh,
        out_type=jax.ShapeDtypeStruct((N_ROIS, OUT_H * OUT_W, C), jnp.float32),
        scratch_types=[
            pltpu.VMEM((RPW, 16), jnp.float32),
            pltpu.VMEM((NWIN, NWIN, C), jnp.float32),
            pltpu.VMEM((NWIN, OUT_W, C), jnp.float32),
            pltpu.VMEM((OUT_H * OUT_W, C), jnp.float32),
            pltpu.SemaphoreType.DMA,
            pltpu.SemaphoreType.DMA,
        ],
        compiler_params=pltpu.CompilerParams(use_tc_tiling_on_sc=False),
    )
    return f(feat2d, rois_pad)


def kernel(input, rois):
    feat2d = jnp.transpose(input, (0, 2, 3, 1)).reshape(B * H, W, C)
    rois_pad = jnp.pad(rois, ((0, 0), (0, 11)))
    out = _roialign_sc(feat2d, rois_pad)
    return jnp.transpose(out, (0, 2, 1)).reshape(N_ROIS, C, OUT_H, OUT_W)


# speedup vs baseline: 1.4978x; 1.4978x over previous
import functools

import jax
import jax.numpy as jnp
from jax import lax
from jax.experimental import pallas as pl
from jax.experimental.pallas import tpu as pltpu
from jax.experimental.pallas import tpu_sc as plsc

OUT_H = 7
OUT_W = 7
SPATIAL_SCALE = 0.0625
SAMPLING_RATIO = 2

B, C, H, W = 4, 256, 64, 64
N_ROIS = 512
NWIN = 17
NWORK = 32
RPW = N_ROIS // NWORK
CCHUNKS = C // 16


def _axis_taps(coord, spacing, size):
    ii = lax.iota(jnp.int32, 16).astype(jnp.float32) + 0.5
    V = coord + ii * spacing
    valid = (V > -1.0) & (V < float(size))
    v = jnp.maximum(V, 0.0)
    l0 = v.astype(jnp.int32)
    cond = l0 >= size - 1
    lo = jnp.where(cond, size - 1, l0)
    hi = jnp.where(cond, size - 1, l0 + 1)
    vf = jnp.where(cond, float(size - 1), v)
    lw = vf - lo.astype(jnp.float32)
    wlo = jnp.where(valid, (1.0 - lw) * 0.5, 0.0)
    whi = jnp.where(valid, lw * 0.5, 0.0)
    return lo, hi, wlo, whi


def _taps(roiv, r):
    row = roiv[r]
    bi = row[0].astype(jnp.int32)
    x1 = row[1] * SPATIAL_SCALE
    y1 = row[2] * SPATIAL_SCALE
    x2 = row[3] * SPATIAL_SCALE
    y2 = row[4] * SPATIAL_SCALE
    sw = jnp.maximum(x2 - x1, 1.0) * (1.0 / (OUT_W * SAMPLING_RATIO))
    sh = jnp.maximum(y2 - y1, 1.0) * (1.0 / (OUT_H * SAMPLING_RATIO))

    lox, hix, wlox, whix = _axis_taps(x1, sw, W)
    loy, hiy, wloy, whiy = _axis_taps(y1, sh, H)

    x0 = jnp.minimum(lox[0], W - NWIN)
    y0 = jnp.minimum(loy[0], H - NWIN)
    return (bi * H + y0, x0,
            lox - x0, hix - x0, loy - y0, hiy - y0,
            wlox, whix, wloy, whiy)


def _body(feat_hbm, rois_hbm, out_hbm, roiv, tile, tmp, acc, sem, osem):
    cid = lax.axis_index("c")
    sid = lax.axis_index("s")
    wid = sid * 2 + cid
    roi0 = wid * RPW
    pltpu.sync_copy(rois_hbm.at[pl.ds(roi0, RPW)], roiv)

    def issue_window(t):
        bh0, x0 = t[0], t[1]
        pltpu.async_copy(
            feat_hbm.at[pl.ds(bh0, NWIN), pl.ds(x0, NWIN)], tile, sem)

    t0 = _taps(roiv, 0)
    issue_window(t0)

    def do_roi(r, t_cur):
        (bh0, x0, jxl, jxh, jyl, jyh, wlox, whix, wloy, whiy) = t_cur

        pltpu.make_async_copy(
            feat_hbm.at[pl.ds(0, NWIN), pl.ds(0, NWIN)], tile, sem).wait()

        for ox in range(OUT_W):
            j0 = jxl[2 * ox]
            j1 = jxh[2 * ox]
            j2 = jxl[2 * ox + 1]
            j3 = jxh[2 * ox + 1]
            w0 = wlox[2 * ox]
            w1 = whix[2 * ox]
            w2 = wlox[2 * ox + 1]
            w3 = whix[2 * ox + 1]

            @plsc.parallel_loop(0, NWIN)
            def yb(y, j0=j0, j1=j1, j2=j2, j3=j3,
                   w0=w0, w1=w1, w2=w2, w3=w3, ox=ox):
                for cc in range(CCHUNKS):
                    sl = pl.ds(cc * 16, 16)
                    tmp[y, ox, sl] = ((tile[y, j0, sl] * w0
                                       + tile[y, j1, sl] * w1)
                                      + (tile[y, j2, sl] * w2
                                         + tile[y, j3, sl] * w3))

        t_next = _taps(roiv, jnp.minimum(r + 1, RPW - 1))

        @pl.when(r + 1 < RPW)
        def _():
            issue_window(t_next)

        ab = r & 1

        @pl.when(r >= 2)
        def _():
            pltpu.make_async_copy(acc.at[ab], out_hbm.at[roi0], osem).wait()

        for oy in range(OUT_H):
            k0 = jyl[2 * oy]
            k1 = jyh[2 * oy]
            k2 = jyl[2 * oy + 1]
            k3 = jyh[2 * oy + 1]
            u0 = wloy[2 * oy]
            u1 = whiy[2 * oy]
            u2 = wloy[2 * oy + 1]
            u3 = whiy[2 * oy + 1]

            @plsc.parallel_loop(0, OUT_W)
            def xb(ox, k0=k0, k1=k1, k2=k2, k3=k3,
                   u0=u0, u1=u1, u2=u2, u3=u3, oy=oy, ab=ab):
                for cc in range(CCHUNKS):
                    sl = pl.ds(cc * 16, 16)
                    acc[ab, oy * OUT_W + ox, sl] = ((tmp[k0, ox, sl] * u0
                                                 + tmp[k1, ox, sl] * u1)
                                                + (tmp[k2, ox, sl] * u2
                                                   + tmp[k3, ox, sl] * u3))

        pltpu.async_copy(acc.at[ab], out_hbm.at[roi0 + r], osem)
        return t_next

    lax.fori_loop(0, RPW, do_roi, t0)
    pltpu.make_async_copy(acc.at[0], out_hbm.at[roi0], osem).wait()
    pltpu.make_async_copy(acc.at[1], out_hbm.at[roi0], osem).wait()


@jax.jit
def _roialign_sc(feat2d, rois_pad):
    mesh = plsc.VectorSubcoreMesh(core_axis_name="c", subcore_axis_name="s")
    f = pl.kernel(
        _body,
        mesh=mesh,
        out_type=jax.ShapeDtypeStruct((N_ROIS, OUT_H * OUT_W, C), jnp.float32),
        scratch_types=[
            pltpu.VMEM((RPW, 16), jnp.float32),
            pltpu.VMEM((NWIN, NWIN, C), jnp.float32),
            pltpu.VMEM((NWIN, OUT_W, C), jnp.float32),
            pltpu.VMEM((2, OUT_H * OUT_W, C), jnp.float32),
            pltpu.SemaphoreType.DMA,
            pltpu.SemaphoreType.DMA,
        ],
        compiler_params=pltpu.CompilerParams(use_tc_tiling_on_sc=False),
    )
    return f(feat2d, rois_pad)


def kernel(input, rois):
    feat2d = jnp.transpose(input, (0, 2, 3, 1)).reshape(B * H, W, C)
    rois_pad = jnp.pad(rois, ((0, 0), (0, 11)))
    out = _roialign_sc(feat2d, rois_pad)
    return jnp.transpose(out, (0, 2, 1)).reshape(N_ROIS, C, OUT_H, OUT_W)
